# Initial kernel scaffold; baseline (speedup 1.0000x reference)
import jax
import jax.numpy as jnp
from jax.experimental import pallas as pl


def kernel(token_logits, sampling_params):
    raise NotImplementedError("write your pallas kernel here")


# speedup vs baseline: 5.3891x; 5.3891x over previous
import functools

import jax
import jax.numpy as jnp
from jax.experimental import pallas as pl
from jax.experimental.pallas import tpu as pltpu

_K = 64
_ROWS = 8
_IGNORED = -3000.0


def _topk_body(x_ref, vals_ref, idxs_ref, scratch_ref):
    rows, vpad = scratch_ref.shape
    scratch_ref[...] = x_ref[...]
    iota = jax.lax.broadcasted_iota(jnp.int32, (rows, vpad), 1)
    lane = jax.lax.broadcasted_iota(jnp.int32, (rows, _K), 1)

    def body(j, carry):
        vals, idxs = carry
        data = scratch_ref[...]
        m = jnp.max(data, axis=1, keepdims=True)
        cand = jnp.where(data == m, iota, -1)
        am = jnp.max(cand, axis=1, keepdims=True)
        scratch_ref[...] = jnp.where(cand == am, -jnp.inf, data)
        vals = jnp.where(lane == j, m, vals)
        idxs = jnp.where(lane == j, am, idxs)
        return vals, idxs

    init = (jnp.full((rows, _K), -jnp.inf, jnp.float32),
            jnp.zeros((rows, _K), jnp.int32))
    vals, idxs = jax.lax.fori_loop(0, _K, body, init)
    vals_ref[...] = vals
    idxs_ref[...] = idxs


def _cumsum_lanes(p):
    b, k = p.shape
    c = p
    sh = 1
    while sh < k:
        z = jnp.zeros((b, sh), p.dtype)
        c = c + jnp.concatenate([z, c[:, : k - sh]], axis=1)
        sh *= 2
    return c


def _sample_body(vals_ref, idxs_ref, sp_ref, out_ref):
    b, k = vals_ref.shape
    v = vals_ref[...]
    idx = idxs_ref[...]
    top_k = sp_ref[:, 0:1]
    top_p = sp_ref[:, 1:2]
    temp = sp_ref[:, 2:3]

    posi = jax.lax.broadcasted_iota(jnp.int32, (b, k), 1)
    pos = posi.astype(jnp.float32)
    sl = jnp.where(pos >= top_k, _IGNORED, v)
    x = sl / temp

    m = jnp.max(x, axis=1, keepdims=True)
    e = jnp.exp(x - m)
    s = jnp.sum(e, axis=1, keepdims=True)
    p = e / s
    c = _cumsum_lanes(p)

    top_p_eff = jnp.maximum(jnp.min(c), top_p)
    pm = (c > top_p_eff) & (posi > 0)
    x2 = jnp.where(pm, _IGNORED, x)

    m2 = jnp.max(x2, axis=1, keepdims=True)
    e2 = jnp.exp(x2 - m2)
    s2 = jnp.sum(e2, axis=1, keepdims=True)
    p2 = e2 / s2
    d = _cumsum_lanes(p2)

    cnt = jnp.sum((d < 0.5).astype(jnp.int32), axis=1, keepdims=True)
    token = jnp.sum(jnp.where(posi == cnt, idx, 0), axis=1, keepdims=True)
    out_ref[...] = token


@jax.jit
def kernel(token_logits, sampling_params):
    b, v = token_logits.shape
    vpad = ((v + 127) // 128) * 128
    x = token_logits
    if vpad != v:
        x = jnp.pad(x, ((0, 0), (0, vpad - v)), constant_values=-jnp.inf)
    rows = _ROWS if b % _ROWS == 0 else 1
    grid = b // rows

    vals, idxs = pl.pallas_call(
        _topk_body,
        grid=(grid,),
        in_specs=[pl.BlockSpec((rows, vpad), lambda i: (i, 0))],
        out_specs=[
            pl.BlockSpec((rows, _K), lambda i: (i, 0)),
            pl.BlockSpec((rows, _K), lambda i: (i, 0)),
        ],
        out_shape=[
            jax.ShapeDtypeStruct((b, _K), jnp.float32),
            jax.ShapeDtypeStruct((b, _K), jnp.int32),
        ],
        scratch_shapes=[pltpu.VMEM((rows, vpad), jnp.float32)],
        compiler_params=pltpu.CompilerParams(
            dimension_semantics=("arbitrary",)),
    )(x)

    token = pl.pallas_call(
        _sample_body,
        out_shape=jax.ShapeDtypeStruct((b, 1), jnp.int32),
    )(vals, idxs, sampling_params)
    return token.reshape(-1)


# speedup vs baseline: 18.6555x; 3.4617x over previous
import functools

import jax
import jax.numpy as jnp
from jax.experimental import pallas as pl
from jax.experimental.pallas import tpu as pltpu

_K = 64
_ROWS = 8
_IGNORED = -3000.0


_LV = 4
_L = 128


def _topk_body(x_ref, vals_ref, idxs_ref):
    rows, w, lanes = x_ref.shape
    w_iota = jax.lax.broadcasted_iota(jnp.int32, (rows, w, lanes), 1)
    l_iota2 = jax.lax.broadcasted_iota(jnp.int32, (rows, lanes), 1)
    lane_k = jax.lax.broadcasted_iota(jnp.int32, (rows, _K), 1)

    dm = x_ref[...]
    heaps = []
    for t in range(_LV):
        mt = jnp.max(dm, axis=1, keepdims=True)
        awt = jnp.max(jnp.where(dm == mt, w_iota, -1), axis=1, keepdims=True)
        heaps.append((mt[:, 0, :], awt[:, 0, :]))
        if t < _LV - 1:
            dm = jnp.where(w_iota == awt, -jnp.inf, dm)

    def body(j, st):
        vals, idxs, curval, curaw, ptr = st
        curi = curaw * lanes + l_iota2
        m = jnp.max(curval, axis=1, keepdims=True)
        ibest = jnp.max(jnp.where(curval == m, curi, -1), axis=1,
                        keepdims=True)
        vals = jnp.where(lane_k == j, m, vals)
        idxs = jnp.where(lane_k == j, ibest, idxs)
        chosen = curi == ibest
        nptr = ptr + chosen.astype(jnp.int32)

        hv = jnp.full_like(curval, -jnp.inf)
        ha = jnp.full_like(curaw, -1)
        for t in range(_LV):
            hv = jnp.where(nptr == t, heaps[t][0], hv)
            ha = jnp.where(nptr == t, heaps[t][1], ha)

        overflow = jnp.any(chosen & (nptr >= _LV))

        def rescan():
            d = x_ref[...]
            ii = w_iota * lanes + jax.lax.broadcasted_iota(
                jnp.int32, (rows, w, lanes), 2)
            m3 = m[:, :, None]
            ib3 = ibest[:, :, None]
            keep = (d < m3) | ((d == m3) & (ii < ib3))
            dx = jnp.where(keep, d, -jnp.inf)
            nv = jnp.max(dx, axis=1, keepdims=True)
            na = jnp.max(jnp.where(dx == nv, w_iota, -1), axis=1,
                         keepdims=True)
            return nv[:, 0, :], na[:, 0, :]

        rv, ra = jax.lax.cond(overflow, rescan, lambda: (hv, ha))
        curval = jnp.where(chosen, rv, curval)
        curaw = jnp.where(chosen, ra, curaw)
        return vals, idxs, curval, curaw, nptr

    init = (jnp.full((rows, _K), -jnp.inf, jnp.float32),
            jnp.zeros((rows, _K), jnp.int32),
            heaps[0][0], heaps[0][1],
            jnp.zeros((rows, lanes), jnp.int32))
    vals, idxs, _, _, _ = jax.lax.fori_loop(0, _K, body, init)
    vals_ref[...] = vals
    idxs_ref[...] = idxs


def _cumsum_lanes(p):
    b, k = p.shape
    c = p
    sh = 1
    while sh < k:
        z = jnp.zeros((b, sh), p.dtype)
        c = c + jnp.concatenate([z, c[:, : k - sh]], axis=1)
        sh *= 2
    return c


def _sample_body(vals_ref, idxs_ref, sp_ref, out_ref):
    b, k = vals_ref.shape
    v = vals_ref[...]
    idx = idxs_ref[...]
    top_k = sp_ref[:, 0:1]
    top_p = sp_ref[:, 1:2]
    temp = sp_ref[:, 2:3]

    posi = jax.lax.broadcasted_iota(jnp.int32, (b, k), 1)
    pos = posi.astype(jnp.float32)
    sl = jnp.where(pos >= top_k, _IGNORED, v)
    x = sl / temp

    m = jnp.max(x, axis=1, keepdims=True)
    e = jnp.exp(x - m)
    s = jnp.sum(e, axis=1, keepdims=True)
    p = e / s
    c = _cumsum_lanes(p)

    top_p_eff = jnp.maximum(jnp.min(c), top_p)
    pm = (c > top_p_eff) & (posi > 0)
    x2 = jnp.where(pm, _IGNORED, x)

    m2 = jnp.max(x2, axis=1, keepdims=True)
    e2 = jnp.exp(x2 - m2)
    s2 = jnp.sum(e2, axis=1, keepdims=True)
    p2 = e2 / s2
    d = _cumsum_lanes(p2)

    cnt = jnp.sum((d < 0.5).astype(jnp.int32), axis=1, keepdims=True)
    token = jnp.sum(jnp.where(posi == cnt, idx, 0), axis=1, keepdims=True)
    out_ref[...] = token


@jax.jit
def kernel(token_logits, sampling_params):
    b, v = token_logits.shape
    vpad = ((v + 127) // 128) * 128
    x = token_logits
    if vpad != v:
        x = jnp.pad(x, ((0, 0), (0, vpad - v)), constant_values=-jnp.inf)
    wdim = vpad // _L
    x = x.reshape(b, wdim, _L)
    rows = _ROWS if b % _ROWS == 0 else 1
    grid = b // rows

    vals, idxs = pl.pallas_call(
        _topk_body,
        grid=(grid,),
        in_specs=[pl.BlockSpec((rows, wdim, _L), lambda i: (i, 0, 0))],
        out_specs=[
            pl.BlockSpec((rows, _K), lambda i: (i, 0)),
            pl.BlockSpec((rows, _K), lambda i: (i, 0)),
        ],
        out_shape=[
            jax.ShapeDtypeStruct((b, _K), jnp.float32),
            jax.ShapeDtypeStruct((b, _K), jnp.int32),
        ],
        compiler_params=pltpu.CompilerParams(
            dimension_semantics=("arbitrary",)),
    )(x)

    token = pl.pallas_call(
        _sample_body,
        out_shape=jax.ShapeDtypeStruct((b, 1), jnp.int32),
    )(vals, idxs, sampling_params)
    return token.reshape(-1)


# speedup vs baseline: 30.0804x; 1.6124x over previous
import functools

import jax
import jax.numpy as jnp
from jax.experimental import pallas as pl
from jax.experimental.pallas import tpu as pltpu

_K = 64
_ROWS = 16
_IGNORED = -3000.0


_LV = 6
_L = 128


def _topk_body(x_ref, vals_ref, idxs_ref):
    rows, w, lanes = x_ref.shape
    w_iota = jax.lax.broadcasted_iota(jnp.int32, (rows, w, lanes), 1)
    l_iota2 = jax.lax.broadcasted_iota(jnp.int32, (rows, lanes), 1)
    lane_k = jax.lax.broadcasted_iota(jnp.int32, (rows, _K), 1)

    dm = x_ref[...]
    heaps = []
    for t in range(_LV):
        mt = jnp.max(dm, axis=1, keepdims=True)
        awt = jnp.max(jnp.where(dm == mt, w_iota, -1), axis=1, keepdims=True)
        heaps.append((mt[:, 0, :], awt[:, 0, :]))
        if t < _LV - 1:
            dm = jnp.where(w_iota == awt, -jnp.inf, dm)

    def select_step(j, vals, idxs, curval, curaw, ptr):
        curi = curaw * lanes + l_iota2
        m = jnp.max(curval, axis=1, keepdims=True)
        ibest = jnp.max(jnp.where(curval == m, curi, -1), axis=1,
                        keepdims=True)
        vals = jnp.where(lane_k == j, m, vals)
        idxs = jnp.where(lane_k == j, ibest, idxs)
        chosen = curi == ibest
        nptr = ptr + chosen.astype(jnp.int32)

        hv = jnp.full_like(curval, -jnp.inf)
        ha = jnp.full_like(curaw, -1)
        for t in range(_LV):
            hv = jnp.where(nptr == t, heaps[t][0], hv)
            ha = jnp.where(nptr == t, heaps[t][1], ha)
        return vals, idxs, chosen, nptr, hv, ha, m, ibest

    def fast_body(j, st):
        vals, idxs, curval, curaw, ptr, ov = st
        vals, idxs, chosen, nptr, hv, ha, _, _ = select_step(
            j, vals, idxs, curval, curaw, ptr)
        ov = jnp.maximum(ov, (chosen & (nptr >= _LV)).astype(jnp.int32))
        curval = jnp.where(chosen, hv, curval)
        curaw = jnp.where(chosen, ha, curaw)
        return vals, idxs, curval, curaw, nptr, ov

    def init_state():
        return (jnp.full((rows, _K), -jnp.inf, jnp.float32),
                jnp.zeros((rows, _K), jnp.int32),
                heaps[0][0], heaps[0][1],
                jnp.zeros((rows, lanes), jnp.int32))

    ov0 = jnp.zeros((rows, lanes), jnp.int32)
    fvals, fidxs, _, _, _, ov = jax.lax.fori_loop(
        0, _K, fast_body, init_state() + (ov0,))

    def slow_path():
        def slow_body(j, st):
            vals, idxs, curval, curaw, ptr = st
            vals, idxs, chosen, nptr, hv, ha, m, ibest = select_step(
                j, vals, idxs, curval, curaw, ptr)
            overflow = jnp.any(chosen & (nptr >= _LV))

            def rescan():
                d = x_ref[...]
                ii = w_iota * lanes + jax.lax.broadcasted_iota(
                    jnp.int32, (rows, w, lanes), 2)
                m3 = m[:, :, None]
                ib3 = ibest[:, :, None]
                keep = (d < m3) | ((d == m3) & (ii < ib3))
                dx = jnp.where(keep, d, -jnp.inf)
                nv = jnp.max(dx, axis=1, keepdims=True)
                na = jnp.max(jnp.where(dx == nv, w_iota, -1), axis=1,
                             keepdims=True)
                return nv[:, 0, :], na[:, 0, :]

            rv, ra = jax.lax.cond(overflow, rescan, lambda: (hv, ha))
            curval = jnp.where(chosen, rv, curval)
            curaw = jnp.where(chosen, ra, curaw)
            return vals, idxs, curval, curaw, nptr

        svals, sidxs, _, _, _ = jax.lax.fori_loop(
            0, _K, slow_body, init_state())
        return svals, sidxs

    vals, idxs = jax.lax.cond(jnp.max(ov) > 0, slow_path,
                              lambda: (fvals, fidxs))
    vals_ref[...] = vals
    idxs_ref[...] = idxs


def _cumsum_lanes(p):
    b, k = p.shape
    c = p
    sh = 1
    while sh < k:
        z = jnp.zeros((b, sh), p.dtype)
        c = c + jnp.concatenate([z, c[:, : k - sh]], axis=1)
        sh *= 2
    return c


def _sample_body(vals_ref, idxs_ref, sp_ref, out_ref):
    b, k = vals_ref.shape
    v = vals_ref[...]
    idx = idxs_ref[...]
    top_k = sp_ref[:, 0:1]
    top_p = sp_ref[:, 1:2]
    temp = sp_ref[:, 2:3]

    posi = jax.lax.broadcasted_iota(jnp.int32, (b, k), 1)
    pos = posi.astype(jnp.float32)
    sl = jnp.where(pos >= top_k, _IGNORED, v)
    x = sl / temp

    m = jnp.max(x, axis=1, keepdims=True)
    e = jnp.exp(x - m)
    s = jnp.sum(e, axis=1, keepdims=True)
    p = e / s
    c = _cumsum_lanes(p)

    top_p_eff = jnp.maximum(jnp.min(c), top_p)
    pm = (c > top_p_eff) & (posi > 0)
    x2 = jnp.where(pm, _IGNORED, x)

    m2 = jnp.max(x2, axis=1, keepdims=True)
    e2 = jnp.exp(x2 - m2)
    s2 = jnp.sum(e2, axis=1, keepdims=True)
    p2 = e2 / s2
    d = _cumsum_lanes(p2)

    cnt = jnp.sum((d < 0.5).astype(jnp.int32), axis=1, keepdims=True)
    token = jnp.sum(jnp.where(posi == cnt, idx, 0), axis=1, keepdims=True)
    out_ref[...] = token


@jax.jit
def kernel(token_logits, sampling_params):
    b, v = token_logits.shape
    vpad = ((v + 127) // 128) * 128
    x = token_logits
    if vpad != v:
        x = jnp.pad(x, ((0, 0), (0, vpad - v)), constant_values=-jnp.inf)
    wdim = vpad // _L
    x = x.reshape(b, wdim, _L)
    rows = _ROWS if b % _ROWS == 0 else 1
    grid = b // rows

    vals, idxs = pl.pallas_call(
        _topk_body,
        grid=(grid,),
        in_specs=[pl.BlockSpec((rows, wdim, _L), lambda i: (i, 0, 0))],
        out_specs=[
            pl.BlockSpec((rows, _K), lambda i: (i, 0)),
            pl.BlockSpec((rows, _K), lambda i: (i, 0)),
        ],
        out_shape=[
            jax.ShapeDtypeStruct((b, _K), jnp.float32),
            jax.ShapeDtypeStruct((b, _K), jnp.int32),
        ],
        compiler_params=pltpu.CompilerParams(
            dimension_semantics=("arbitrary",)),
    )(x)

    token = pl.pallas_call(
        _sample_body,
        out_shape=jax.ShapeDtypeStruct((b, 1), jnp.int32),
    )(vals, idxs, sampling_params)
    return token.reshape(-1)


# speedup vs baseline: 31.0292x; 1.0315x over previous
import functools

import jax
import jax.numpy as jnp
from jax.experimental import pallas as pl
from jax.experimental.pallas import tpu as pltpu

_K = 64
_ROWS = 16
_IGNORED = -3000.0


_LV = 6
_L = 128


def _topk_body(x_ref, vals_ref, idxs_ref):
    rows, w, lanes = x_ref.shape
    w_iota = jax.lax.broadcasted_iota(jnp.int32, (rows, w, lanes), 1)
    l_iota2 = jax.lax.broadcasted_iota(jnp.int32, (rows, lanes), 1)
    lane_k = jax.lax.broadcasted_iota(jnp.int32, (rows, _K), 1)

    dm = x_ref[...]
    heaps = []
    for t in range(_LV):
        mt = jnp.max(dm, axis=1, keepdims=True)
        awt = jnp.max(jnp.where(dm == mt, w_iota, -1), axis=1, keepdims=True)
        heaps.append((mt[:, 0, :], awt[:, 0, :]))
        if t < _LV - 1:
            dm = jnp.where(w_iota == awt, -jnp.inf, dm)

    def select_step(j, vals, idxs, curval, curaw, ptr):
        curi = curaw * lanes + l_iota2
        m = jnp.max(curval, axis=1, keepdims=True)
        ibest = jnp.max(jnp.where(curval == m, curi, -1), axis=1,
                        keepdims=True)
        vals = jnp.where(lane_k == j, m, vals)
        idxs = jnp.where(lane_k == j, ibest, idxs)
        chosen = curi == ibest
        nptr = ptr + chosen.astype(jnp.int32)

        hv = jnp.full_like(curval, -jnp.inf)
        ha = jnp.full_like(curaw, -1)
        for t in range(_LV):
            hv = jnp.where(nptr == t, heaps[t][0], hv)
            ha = jnp.where(nptr == t, heaps[t][1], ha)
        return vals, idxs, chosen, nptr, hv, ha, m, ibest

    def fast_body(j, st):
        vals, idxs, curval, curaw, ptr = st
        vals, idxs, chosen, nptr, hv, ha, _, _ = select_step(
            j, vals, idxs, curval, curaw, ptr)
        curval = jnp.where(chosen, hv, curval)
        curaw = jnp.where(chosen, ha, curaw)
        return vals, idxs, curval, curaw, nptr

    def init_state():
        return (jnp.full((rows, _K), -jnp.inf, jnp.float32),
                jnp.zeros((rows, _K), jnp.int32),
                heaps[0][0], heaps[0][1],
                jnp.zeros((rows, lanes), jnp.int32))

    fvals, fidxs, _, _, fptr = jax.lax.fori_loop(
        0, _K, fast_body, init_state())

    def slow_path():
        def slow_body(j, st):
            vals, idxs, curval, curaw, ptr = st
            vals, idxs, chosen, nptr, hv, ha, m, ibest = select_step(
                j, vals, idxs, curval, curaw, ptr)
            overflow = jnp.any(chosen & (nptr >= _LV))

            def rescan():
                d = x_ref[...]
                ii = w_iota * lanes + jax.lax.broadcasted_iota(
                    jnp.int32, (rows, w, lanes), 2)
                m3 = m[:, :, None]
                ib3 = ibest[:, :, None]
                keep = (d < m3) | ((d == m3) & (ii < ib3))
                dx = jnp.where(keep, d, -jnp.inf)
                nv = jnp.max(dx, axis=1, keepdims=True)
                na = jnp.max(jnp.where(dx == nv, w_iota, -1), axis=1,
                             keepdims=True)
                return nv[:, 0, :], na[:, 0, :]

            rv, ra = jax.lax.cond(overflow, rescan, lambda: (hv, ha))
            curval = jnp.where(chosen, rv, curval)
            curaw = jnp.where(chosen, ra, curaw)
            return vals, idxs, curval, curaw, nptr

        svals, sidxs, _, _, _ = jax.lax.fori_loop(
            0, _K, slow_body, init_state())
        return svals, sidxs

    vals, idxs = jax.lax.cond(jnp.max(fptr) >= _LV, slow_path,
                              lambda: (fvals, fidxs))
    vals_ref[...] = vals
    idxs_ref[...] = idxs


def _cumsum_lanes(p):
    b, k = p.shape
    c = p
    sh = 1
    while sh < k:
        z = jnp.zeros((b, sh), p.dtype)
        c = c + jnp.concatenate([z, c[:, : k - sh]], axis=1)
        sh *= 2
    return c


def _sample_body(vals_ref, idxs_ref, sp_ref, out_ref):
    b, k = vals_ref.shape
    v = vals_ref[...]
    idx = idxs_ref[...]
    top_k = sp_ref[:, 0:1]
    top_p = sp_ref[:, 1:2]
    temp = sp_ref[:, 2:3]

    posi = jax.lax.broadcasted_iota(jnp.int32, (b, k), 1)
    pos = posi.astype(jnp.float32)
    sl = jnp.where(pos >= top_k, _IGNORED, v)
    x = sl / temp

    m = jnp.max(x, axis=1, keepdims=True)
    e = jnp.exp(x - m)
    s = jnp.sum(e, axis=1, keepdims=True)
    p = e / s
    c = _cumsum_lanes(p)

    top_p_eff = jnp.maximum(jnp.min(c), top_p)
    pm = (c > top_p_eff) & (posi > 0)
    x2 = jnp.where(pm, _IGNORED, x)

    m2 = jnp.max(x2, axis=1, keepdims=True)
    e2 = jnp.exp(x2 - m2)
    s2 = jnp.sum(e2, axis=1, keepdims=True)
    p2 = e2 / s2
    d = _cumsum_lanes(p2)

    cnt = jnp.sum((d < 0.5).astype(jnp.int32), axis=1, keepdims=True)
    token = jnp.sum(jnp.where(posi == cnt, idx, 0), axis=1, keepdims=True)
    out_ref[...] = token


@jax.jit
def kernel(token_logits, sampling_params):
    b, v = token_logits.shape
    vpad = ((v + 127) // 128) * 128
    x = token_logits
    if vpad != v:
        x = jnp.pad(x, ((0, 0), (0, vpad - v)), constant_values=-jnp.inf)
    wdim = vpad // _L
    x = x.reshape(b, wdim, _L)
    rows = _ROWS if b % _ROWS == 0 else 1
    grid = b // rows

    vals, idxs = pl.pallas_call(
        _topk_body,
        grid=(grid,),
        in_specs=[pl.BlockSpec((rows, wdim, _L), lambda i: (i, 0, 0))],
        out_specs=[
            pl.BlockSpec((rows, _K), lambda i: (i, 0)),
            pl.BlockSpec((rows, _K), lambda i: (i, 0)),
        ],
        out_shape=[
            jax.ShapeDtypeStruct((b, _K), jnp.float32),
            jax.ShapeDtypeStruct((b, _K), jnp.int32),
        ],
        compiler_params=pltpu.CompilerParams(
            dimension_semantics=("arbitrary",)),
    )(x)

    token = pl.pallas_call(
        _sample_body,
        out_shape=jax.ShapeDtypeStruct((b, 1), jnp.int32),
    )(vals, idxs, sampling_params)
    return token.reshape(-1)


# speedup vs baseline: 32.1201x; 1.0352x over previous
import functools

import jax
import jax.numpy as jnp
from jax.experimental import pallas as pl
from jax.experimental.pallas import tpu as pltpu

_K = 64
_ROWS = 16
_IGNORED = -3000.0


_LV = 6
_L = 128


def _topk_body(x_ref, vals_ref, idxs_ref):
    rows, w, lanes = x_ref.shape
    w_iota = jax.lax.broadcasted_iota(jnp.int32, (rows, w, lanes), 1)
    l_iota2 = jax.lax.broadcasted_iota(jnp.int32, (rows, lanes), 1)
    lane_k = jax.lax.broadcasted_iota(jnp.int32, (rows, _K), 1)

    dm = x_ref[...]
    heaps = []
    for t in range(_LV):
        mt = jnp.max(dm, axis=1, keepdims=True)
        awt = jnp.max(jnp.where(dm == mt, w_iota, -1), axis=1, keepdims=True)
        heaps.append((mt[:, 0, :], awt[:, 0, :]))
        if t < _LV - 1:
            dm = jnp.where(w_iota == awt, -jnp.inf, dm)

    def select_step(j, vals, idxs, curval, curaw, ptr):
        curi = curaw * lanes + l_iota2
        m = jnp.max(curval, axis=1, keepdims=True)
        ibest = jnp.max(jnp.where(curval == m, curi, -1), axis=1,
                        keepdims=True)
        vals = jnp.where(lane_k == j, m, vals)
        idxs = jnp.where(lane_k == j, ibest, idxs)
        chosen = curi == ibest
        nptr = ptr + chosen.astype(jnp.int32)

        hv = jnp.full_like(curval, -jnp.inf)
        ha = jnp.full_like(curaw, -1)
        for t in range(_LV):
            hv = jnp.where(nptr == t, heaps[t][0], hv)
            ha = jnp.where(nptr == t, heaps[t][1], ha)
        return vals, idxs, chosen, nptr, hv, ha, m, ibest

    def fast_body(j, st):
        vals, idxs, curval, curaw, ptr = st
        vals, idxs, chosen, nptr, hv, ha, _, _ = select_step(
            j, vals, idxs, curval, curaw, ptr)
        curval = jnp.where(chosen, hv, curval)
        curaw = jnp.where(chosen, ha, curaw)
        return vals, idxs, curval, curaw, nptr

    def init_state():
        return (jnp.full((rows, _K), -jnp.inf, jnp.float32),
                jnp.zeros((rows, _K), jnp.int32),
                heaps[0][0], heaps[0][1],
                jnp.zeros((rows, lanes), jnp.int32))

    fvals, fidxs, _, _, fptr = jax.lax.fori_loop(
        0, _K, fast_body, init_state(), unroll=4)

    def slow_path():
        def slow_body(j, st):
            vals, idxs, curval, curaw, ptr = st
            vals, idxs, chosen, nptr, hv, ha, m, ibest = select_step(
                j, vals, idxs, curval, curaw, ptr)
            overflow = jnp.any(chosen & (nptr >= _LV))

            def rescan():
                d = x_ref[...]
                ii = w_iota * lanes + jax.lax.broadcasted_iota(
                    jnp.int32, (rows, w, lanes), 2)
                m3 = m[:, :, None]
                ib3 = ibest[:, :, None]
                keep = (d < m3) | ((d == m3) & (ii < ib3))
                dx = jnp.where(keep, d, -jnp.inf)
                nv = jnp.max(dx, axis=1, keepdims=True)
                na = jnp.max(jnp.where(dx == nv, w_iota, -1), axis=1,
                             keepdims=True)
                return nv[:, 0, :], na[:, 0, :]

            rv, ra = jax.lax.cond(overflow, rescan, lambda: (hv, ha))
            curval = jnp.where(chosen, rv, curval)
            curaw = jnp.where(chosen, ra, curaw)
            return vals, idxs, curval, curaw, nptr

        svals, sidxs, _, _, _ = jax.lax.fori_loop(
            0, _K, slow_body, init_state())
        return svals, sidxs

    vals, idxs = jax.lax.cond(jnp.max(fptr) >= _LV, slow_path,
                              lambda: (fvals, fidxs))
    vals_ref[...] = vals
    idxs_ref[...] = idxs


def _cumsum_lanes(p):
    b, k = p.shape
    c = p
    sh = 1
    while sh < k:
        z = jnp.zeros((b, sh), p.dtype)
        c = c + jnp.concatenate([z, c[:, : k - sh]], axis=1)
        sh *= 2
    return c


def _sample_body(vals_ref, idxs_ref, sp_ref, out_ref):
    b, k = vals_ref.shape
    v = vals_ref[...]
    idx = idxs_ref[...]
    top_k = sp_ref[:, 0:1]
    top_p = sp_ref[:, 1:2]
    temp = sp_ref[:, 2:3]

    posi = jax.lax.broadcasted_iota(jnp.int32, (b, k), 1)
    pos = posi.astype(jnp.float32)
    sl = jnp.where(pos >= top_k, _IGNORED, v)
    x = sl / temp

    m = jnp.max(x, axis=1, keepdims=True)
    e = jnp.exp(x - m)
    s = jnp.sum(e, axis=1, keepdims=True)
    p = e / s
    c = _cumsum_lanes(p)

    top_p_eff = jnp.maximum(jnp.min(c), top_p)
    pm = (c > top_p_eff) & (posi > 0)
    x2 = jnp.where(pm, _IGNORED, x)

    m2 = jnp.max(x2, axis=1, keepdims=True)
    e2 = jnp.exp(x2 - m2)
    s2 = jnp.sum(e2, axis=1, keepdims=True)
    p2 = e2 / s2
    d = _cumsum_lanes(p2)

    cnt = jnp.sum((d < 0.5).astype(jnp.int32), axis=1, keepdims=True)
    token = jnp.sum(jnp.where(posi == cnt, idx, 0), axis=1, keepdims=True)
    out_ref[...] = token


@jax.jit
def kernel(token_logits, sampling_params):
    b, v = token_logits.shape
    vpad = ((v + 127) // 128) * 128
    x = token_logits
    if vpad != v:
        x = jnp.pad(x, ((0, 0), (0, vpad - v)), constant_values=-jnp.inf)
    wdim = vpad // _L
    x = x.reshape(b, wdim, _L)
    rows = _ROWS if b % _ROWS == 0 else 1
    grid = b // rows

    vals, idxs = pl.pallas_call(
        _topk_body,
        grid=(grid,),
        in_specs=[pl.BlockSpec((rows, wdim, _L), lambda i: (i, 0, 0))],
        out_specs=[
            pl.BlockSpec((rows, _K), lambda i: (i, 0)),
            pl.BlockSpec((rows, _K), lambda i: (i, 0)),
        ],
        out_shape=[
            jax.ShapeDtypeStruct((b, _K), jnp.float32),
            jax.ShapeDtypeStruct((b, _K), jnp.int32),
        ],
        compiler_params=pltpu.CompilerParams(
            dimension_semantics=("arbitrary",)),
    )(x)

    token = pl.pallas_call(
        _sample_body,
        out_shape=jax.ShapeDtypeStruct((b, 1), jnp.int32),
    )(vals, idxs, sampling_params)
    return token.reshape(-1)
